# baseline (speedup 1.0000x reference)
import functools

import jax
import jax.numpy as jnp
from jax import lax
from jax.experimental import pallas as pl
from jax.experimental.pallas import tpu as pltpu
from jax.experimental.pallas import tpu_sc as plsc

B = 4096
F = 26
N_USER = 17
N_TARGET = 9
L = 50
D = 64
MID = 128
ROWS = F + L

NC = 2
NS = 16
NW = NC * NS
SPW = B // NW
NBUF = 2


GRP = 4
GPW = SPW // GRP
FR = GRP * F
WG = GRP * L
WRA = 104
WRB = WG - WRA
_WSEG = {
    0: [(0, 0, 50)],
    1: [(0, 50, 50)],
    2: [(0, 100, 4), (1, 0, 46)],
    3: [(1, 46, 50)],
}


def _sc_gather_pool(idx_f, idx_w, scidx, table):
    mesh = plsc.VectorSubcoreMesh(core_axis_name="c", subcore_axis_name="s")

    @functools.partial(
        pl.kernel,
        mesh=mesh,
        compiler_params=pltpu.CompilerParams(use_tc_tiling_on_sc=False),
        out_type=[
            jax.ShapeDtypeStruct((B * F, D), jnp.float32),
            jax.ShapeDtypeStruct((B, 2 * D), jnp.float32),
        ],
        scratch_types=(
            [
                pltpu.VMEM((SPW * F,), jnp.int32),
                pltpu.VMEM((SPW * L,), jnp.int32),
                pltpu.VMEM((GPW, FR), jnp.int32),
                pltpu.VMEM((SPW, 2 * D), jnp.float32),
            ]
            + [pltpu.VMEM((FR, D), jnp.float32) for _ in range(NBUF)]
            + [pltpu.VMEM((WRA, D), jnp.float32) for _ in range(NBUF)]
            + [pltpu.VMEM((WRB, D), jnp.float32) for _ in range(NBUF)]
            + [pltpu.SemaphoreType.DMA for _ in range(4 * NBUF)]
        ),
    )
    def k(idxf_hbm, idxw_hbm, scidx_hbm, table_hbm, emb_out, pooled_out,
          idxf_v, idxw_v, scidx_v, pooled_v, *rest):
        fbufs = rest[:NBUF]
        wabufs = rest[NBUF:2 * NBUF]
        wbbufs = rest[2 * NBUF:3 * NBUF]
        gsems = rest[3 * NBUF:6 * NBUF]
        osems = rest[6 * NBUF:7 * NBUF]
        wid = lax.axis_index("c") * NS + lax.axis_index("s")
        pltpu.sync_copy(idxf_hbm.at[pl.ds(wid * (SPW * F), SPW * F)], idxf_v)
        pltpu.sync_copy(idxw_hbm.at[pl.ds(wid * (SPW * L), SPW * L)], idxw_v)
        pltpu.sync_copy(scidx_hbm.at[pl.ds(wid * GPW, GPW)], scidx_v)

        def start_f(gr, b):
            pltpu.async_copy(table_hbm.at[idxf_v.at[pl.ds(gr * FR, FR)]],
                             fbufs[b], gsems[3 * b])

        def start_w(gr, b):
            pltpu.async_copy(table_hbm.at[idxw_v.at[pl.ds(gr * WG, WRA)]],
                             wabufs[b], gsems[3 * b + 1])
            pltpu.async_copy(
                table_hbm.at[idxw_v.at[pl.ds(gr * WG + WRA, WRB)]],
                wbbufs[b], gsems[3 * b + 2])

        for b in range(NBUF):
            start_f(b, b)
            start_w(b, b)

        def step(i, carry):
            for b in range(NBUF):
                gr = i * NBUF + b
                pltpu.make_async_copy(
                    table_hbm.at[idxf_v.at[pl.ds(gr * FR, FR)]], fbufs[b],
                    gsems[3 * b]).wait()
                out_cp = pltpu.async_copy(
                    fbufs[b], emb_out.at[scidx_v.at[gr]], osems[b])
                pltpu.make_async_copy(
                    table_hbm.at[idxw_v.at[pl.ds(gr * WG, WRA)]], wabufs[b],
                    gsems[3 * b + 1]).wait()
                pltpu.make_async_copy(
                    table_hbm.at[idxw_v.at[pl.ds(gr * WG + WRA, WRB)]],
                    wbbufs[b], gsems[3 * b + 2]).wait()
                halves = (wabufs[b], wbbufs[b])
                for j in range(GRP):
                    g = gr * GRP + j
                    accs = [jnp.zeros((16,), jnp.float32) for _ in range(8)]
                    n = 0
                    for hb, st, cnt in _WSEG[j]:
                        wb = halves[hb]
                        for r in range(st, st + cnt):
                            for c in range(4):
                                a = (n % 2) * 4 + c
                                accs[a] = accs[a] + wb[r, pl.ds(c * 16, 16)]
                            n += 1
                    for c in range(4):
                        m = (accs[c] + accs[c + 4]) * (1.0 / L)
                        pooled_v[g, pl.ds(c * 16, 16)] = m
                        pooled_v[g, pl.ds(D + c * 16, 16)] = m
                out_cp.wait()

                @pl.when(gr + NBUF < GPW)
                def _():
                    start_f(gr + NBUF, b)
                    start_w(gr + NBUF, b)
            return carry

        lax.fori_loop(0, GPW // NBUF, step, 0)
        pltpu.sync_copy(pooled_v, pooled_out.at[pl.ds(wid * SPW, SPW)])

    return k(idx_f, idx_w, scidx, table)


def _tc_linearize_table(table_t):
    V = 100000
    CB = 6400

    def body(t_ref, out_ref):
        t = jnp.transpose(t_ref[...], (1, 0))
        out_ref[...] = jnp.concatenate([t, t], axis=1)

    return pl.pallas_call(
        body,
        grid=(pl.cdiv(V, CB),),
        in_specs=[pl.BlockSpec((D, CB), lambda i: (0, i))],
        out_specs=pl.BlockSpec((CB, 2 * D), lambda i: (i, 0)),
        out_shape=jax.ShapeDtypeStruct((V, 2 * D), jnp.float32),
    )(table_t)


def _tc_dense(emb3, pooled, wbig, wc, bc, wl2, c0):
    BB = 1024
    UD = (N_USER + 1) * D
    TD = N_TARGET * D

    def body(emb_ref, pooled_ref, wbig_ref, wc_ref, bc_ref, wl2_ref, c0_ref,
             out_ref):
        e = emb_ref[...]
        p = pooled_ref[...][:, :D]
        user = jnp.concatenate(
            [e[c] for c in range(8)] + [e[8][:, :D]], axis=1)
        target = jnp.concatenate(
            [e[8][:, D:]] + [e[c] for c in range(9, 13)], axis=1)
        ufl = jnp.concatenate([user, p], axis=1)
        a = jnp.dot(ufl.astype(jnp.bfloat16), wbig_ref[...],
                    preferred_element_type=jnp.float32)
        inter = jnp.sum(a * target, axis=1, keepdims=True)
        dnn_in = jnp.concatenate([target, p], axis=1)
        h = jnp.maximum(
            jnp.dot(dnn_in.astype(jnp.bfloat16), wc_ref[...],
                    preferred_element_type=jnp.float32) + bc_ref[...], 0.0)
        part2 = jnp.sum(h * wl2_ref[...], axis=1, keepdims=True)
        out_ref[...] = inter + part2 + c0_ref[0, 0]

    return pl.pallas_call(
        body,
        grid=(B // BB,),
        in_specs=[
            pl.BlockSpec((F // 2, BB, 2 * D), lambda i: (0, i, 0)),
            pl.BlockSpec((BB, 2 * D), lambda i: (i, 0)),
            pl.BlockSpec((UD, TD), lambda i: (0, 0)),
            pl.BlockSpec(((N_TARGET + 1) * D, MID), lambda i: (0, 0)),
            pl.BlockSpec((1, MID), lambda i: (0, 0)),
            pl.BlockSpec((1, MID), lambda i: (0, 0)),
            pl.BlockSpec((1, 1), lambda i: (0, 0)),
        ],
        out_specs=pl.BlockSpec((BB, 1), lambda i: (i, 0)),
        out_shape=jax.ShapeDtypeStruct((B, 1), jnp.float32),
    )(emb3, pooled, wbig, wc, bc, wl2, c0)


def kernel(fids, watched_fids, table, W_c, b_c, W_l, b_l):
    idx_f = (fids.astype(jnp.int32) * 2).reshape(B * F)
    idx_w = (watched_fids.astype(jnp.int32) * 2).reshape(B * L)
    table_lin = _tc_linearize_table(table.T).reshape(200000, D)
    js = jnp.arange(GRP)[:, None]
    ss = jnp.arange(F)[None, :]
    grs = jnp.arange(B // GRP)[:, None]
    sc_in_grp = ((ss // 2) * (2 * B) + 2 * js + ss % 2).reshape(1, FR)
    scidx = (sc_in_grp + 2 * GRP * grs).astype(jnp.int32)
    emb_flat, pooled = _sc_gather_pool(idx_f, idx_w, scidx, table_lin)
    emb3 = emb_flat.reshape(F // 2, B, 2 * D)
    w1 = W_l[:N_TARGET * (N_USER + 1), 0].reshape(N_TARGET, N_USER + 1)
    eye = jnp.eye(D, dtype=jnp.float32)
    wbig = (w1.T[:, None, :, None] * eye[None, :, None, :]).reshape(
        (N_USER + 1) * D, N_TARGET * D).astype(jnp.bfloat16)
    wl2 = W_l[N_TARGET * (N_USER + 1):, 0].reshape(1, MID)
    return _tc_dense(emb3, pooled, wbig, W_c.astype(jnp.bfloat16),
                     b_c.reshape(1, MID), wl2, b_l.reshape(1, 1))


# speedup vs baseline: 1.0242x; 1.0242x over previous
import functools

import jax
import jax.numpy as jnp
from jax import lax
from jax.experimental import pallas as pl
from jax.experimental.pallas import tpu as pltpu
from jax.experimental.pallas import tpu_sc as plsc

B = 4096
F = 26
N_USER = 17
N_TARGET = 9
L = 50
D = 64
MID = 128
ROWS = F + L

NC = 2
NS = 16
NW = NC * NS
SPW = B // NW
NBUF = 2


GRP = 4
GPW = SPW // GRP
FR = GRP * F
WG = GRP * L
WRA = 104
WRB = WG - WRA
_WSEG = {
    0: [(0, 0, 50)],
    1: [(0, 50, 50)],
    2: [(0, 100, 4), (1, 0, 46)],
    3: [(1, 46, 50)],
}


def _sc_gather_pool(idx_f, idx_w, scidx, table):
    mesh = plsc.VectorSubcoreMesh(core_axis_name="c", subcore_axis_name="s")

    @functools.partial(
        pl.kernel,
        mesh=mesh,
        compiler_params=pltpu.CompilerParams(use_tc_tiling_on_sc=False),
        out_type=[
            jax.ShapeDtypeStruct((B * F, D), jnp.float32),
            jax.ShapeDtypeStruct((B, 2 * D), jnp.float32),
        ],
        scratch_types=(
            [
                pltpu.VMEM((SPW * F,), jnp.int32),
                pltpu.VMEM((SPW * L,), jnp.int32),
                pltpu.VMEM((GPW, FR), jnp.int32),
                pltpu.VMEM((SPW, 2 * D), jnp.float32),
            ]
            + [pltpu.VMEM((FR, D), jnp.float32) for _ in range(NBUF)]
            + [pltpu.VMEM((WRA, D), jnp.float32) for _ in range(NBUF)]
            + [pltpu.VMEM((WRB, D), jnp.float32) for _ in range(NBUF)]
            + [pltpu.SemaphoreType.DMA for _ in range(4 * NBUF)]
        ),
    )
    def k(idxf_hbm, idxw_hbm, scidx_hbm, table_hbm, emb_out, pooled_out,
          idxf_v, idxw_v, scidx_v, pooled_v, *rest):
        fbufs = rest[:NBUF]
        wabufs = rest[NBUF:2 * NBUF]
        wbbufs = rest[2 * NBUF:3 * NBUF]
        gsems = rest[3 * NBUF:6 * NBUF]
        osems = rest[6 * NBUF:7 * NBUF]
        wid = lax.axis_index("c") * NS + lax.axis_index("s")
        pltpu.sync_copy(idxf_hbm.at[pl.ds(wid * (SPW * F), SPW * F)], idxf_v)
        pltpu.sync_copy(idxw_hbm.at[pl.ds(wid * (SPW * L), SPW * L)], idxw_v)
        pltpu.sync_copy(scidx_hbm.at[pl.ds(wid * GPW, GPW)], scidx_v)

        def start_f(gr, b):
            pltpu.async_copy(table_hbm.at[idxf_v.at[pl.ds(gr * FR, FR)]],
                             fbufs[b], gsems[3 * b])

        def start_w(gr, b):
            pltpu.async_copy(table_hbm.at[idxw_v.at[pl.ds(gr * WG, WRA)]],
                             wabufs[b], gsems[3 * b + 1])
            pltpu.async_copy(
                table_hbm.at[idxw_v.at[pl.ds(gr * WG + WRA, WRB)]],
                wbbufs[b], gsems[3 * b + 2])

        for b in range(NBUF):
            start_f(b, b)
            start_w(b, b)

        def step(i, carry):
            for b in range(NBUF):
                gr = i * NBUF + b
                pltpu.make_async_copy(
                    table_hbm.at[idxf_v.at[pl.ds(gr * FR, FR)]], fbufs[b],
                    gsems[3 * b]).wait()
                out_cp = pltpu.async_copy(
                    fbufs[b], emb_out.at[scidx_v.at[gr]], osems[b])
                pltpu.make_async_copy(
                    table_hbm.at[idxw_v.at[pl.ds(gr * WG, WRA)]], wabufs[b],
                    gsems[3 * b + 1]).wait()
                pltpu.make_async_copy(
                    table_hbm.at[idxw_v.at[pl.ds(gr * WG + WRA, WRB)]],
                    wbbufs[b], gsems[3 * b + 2]).wait()
                halves = (wabufs[b], wbbufs[b])
                for j in range(GRP):
                    g = gr * GRP + j
                    accs = [jnp.zeros((16,), jnp.float32) for _ in range(8)]
                    n = 0
                    for hb, st, cnt in _WSEG[j]:
                        wb = halves[hb]
                        for r in range(st, st + cnt):
                            for c in range(4):
                                a = (n % 2) * 4 + c
                                accs[a] = accs[a] + wb[r, pl.ds(c * 16, 16)]
                            n += 1
                    for c in range(4):
                        m = (accs[c] + accs[c + 4]) * (1.0 / L)
                        pooled_v[g, pl.ds(c * 16, 16)] = m
                        pooled_v[g, pl.ds(D + c * 16, 16)] = m
                out_cp.wait()

                @pl.when(gr + NBUF < GPW)
                def _():
                    start_f(gr + NBUF, b)
                    start_w(gr + NBUF, b)
            return carry

        lax.fori_loop(0, GPW // NBUF, step, 0)
        pltpu.sync_copy(pooled_v, pooled_out.at[pl.ds(wid * SPW, SPW)])

    return k(idx_f, idx_w, scidx, table)


def _tc_linearize_table(table_t):
    V = 100000
    CB = 12800

    def body(t_ref, out_ref):
        t = jnp.transpose(t_ref[...], (1, 0))
        out_ref[...] = jnp.concatenate([t, t], axis=1)

    return pl.pallas_call(
        body,
        grid=(pl.cdiv(V, CB),),
        in_specs=[pl.BlockSpec((D, CB), lambda i: (0, i))],
        out_specs=pl.BlockSpec((CB, 2 * D), lambda i: (i, 0)),
        out_shape=jax.ShapeDtypeStruct((V, 2 * D), jnp.float32),
    )(table_t)


def _tc_dense(emb3, pooled, wbig, wc, bc, wl2, c0):
    BB = 1024
    UD = (N_USER + 1) * D
    TD = N_TARGET * D

    def body(emb_ref, pooled_ref, wbig_ref, wc_ref, bc_ref, wl2_ref, c0_ref,
             out_ref):
        e = emb_ref[...]
        p = pooled_ref[...][:, :D]
        user = jnp.concatenate(
            [e[c] for c in range(8)] + [e[8][:, :D]], axis=1)
        target = jnp.concatenate(
            [e[8][:, D:]] + [e[c] for c in range(9, 13)], axis=1)
        ufl = jnp.concatenate([user, p], axis=1)
        a = jnp.dot(ufl.astype(jnp.bfloat16), wbig_ref[...],
                    preferred_element_type=jnp.float32)
        inter = jnp.sum(a * target, axis=1, keepdims=True)
        dnn_in = jnp.concatenate([target, p], axis=1)
        h = jnp.maximum(
            jnp.dot(dnn_in.astype(jnp.bfloat16), wc_ref[...],
                    preferred_element_type=jnp.float32) + bc_ref[...], 0.0)
        part2 = jnp.sum(h * wl2_ref[...], axis=1, keepdims=True)
        out_ref[...] = inter + part2 + c0_ref[0, 0]

    return pl.pallas_call(
        body,
        grid=(B // BB,),
        in_specs=[
            pl.BlockSpec((F // 2, BB, 2 * D), lambda i: (0, i, 0)),
            pl.BlockSpec((BB, 2 * D), lambda i: (i, 0)),
            pl.BlockSpec((UD, TD), lambda i: (0, 0)),
            pl.BlockSpec(((N_TARGET + 1) * D, MID), lambda i: (0, 0)),
            pl.BlockSpec((1, MID), lambda i: (0, 0)),
            pl.BlockSpec((1, MID), lambda i: (0, 0)),
            pl.BlockSpec((1, 1), lambda i: (0, 0)),
        ],
        out_specs=pl.BlockSpec((BB, 1), lambda i: (i, 0)),
        out_shape=jax.ShapeDtypeStruct((B, 1), jnp.float32),
    )(emb3, pooled, wbig, wc, bc, wl2, c0)


def kernel(fids, watched_fids, table, W_c, b_c, W_l, b_l):
    idx_f = (fids.astype(jnp.int32) * 2).reshape(B * F)
    idx_w = (watched_fids.astype(jnp.int32) * 2).reshape(B * L)
    table_lin = _tc_linearize_table(table.T).reshape(200000, D)
    js = jnp.arange(GRP)[:, None]
    ss = jnp.arange(F)[None, :]
    grs = jnp.arange(B // GRP)[:, None]
    sc_in_grp = ((ss // 2) * (2 * B) + 2 * js + ss % 2).reshape(1, FR)
    scidx = (sc_in_grp + 2 * GRP * grs).astype(jnp.int32)
    emb_flat, pooled = _sc_gather_pool(idx_f, idx_w, scidx, table_lin)
    emb3 = emb_flat.reshape(F // 2, B, 2 * D)
    w1 = W_l[:N_TARGET * (N_USER + 1), 0].reshape(N_TARGET, N_USER + 1)
    eye = jnp.eye(D, dtype=jnp.float32)
    wbig = (w1.T[:, None, :, None] * eye[None, :, None, :]).reshape(
        (N_USER + 1) * D, N_TARGET * D).astype(jnp.bfloat16)
    wl2 = W_l[N_TARGET * (N_USER + 1):, 0].reshape(1, MID)
    return _tc_dense(emb3, pooled, wbig, W_c.astype(jnp.bfloat16),
                     b_c.reshape(1, MID), wl2, b_l.reshape(1, 1))
